# baseline (speedup 1.0000x reference)
import jax
import jax.numpy as jnp
from jax.experimental import pallas as pl
from jax.experimental.pallas import tpu as pltpu

E = 16
H = 1024
FFN = 2048
TOP_K = 2
T = 128
FH = FFN // 2
HH = H // 2


def _moe_body(x_ref, wr_ref, w1_ref, b1_ref, w2_ref, b2_ref,
              out_ref, probs_ref, topk_ref, c_ref, h_ref):
    e = pl.program_id(0)
    s = pl.program_id(1)

    @pl.when((e == 0) & (s == 0))
    def _router():
        xx = x_ref[...]
        g = jax.lax.dot_general(
            xx, wr_ref[...],
            dimension_numbers=(((1,), (1,)), ((), ())),
            preferred_element_type=jnp.float32)
        m = jnp.max(g, axis=-1, keepdims=True)
        ex = jnp.exp(g - m)
        p = ex / jnp.sum(ex, axis=-1, keepdims=True)
        probs_ref[...] = p
        cols = jax.lax.broadcasted_iota(jnp.int32, (T, E), 1)
        i1 = jnp.argmax(p, axis=-1)
        p1 = jnp.max(p, axis=-1)
        pm = jnp.where(cols == i1[:, None], -1.0, p)
        i2 = jnp.argmax(pm, axis=-1)
        p2 = jnp.max(pm, axis=-1)
        sm = p1 + p2 + 1e-9
        p1n = (p1 / sm)[:, None]
        p2n = (p2 / sm)[:, None]
        topk_ref[...] = jnp.concatenate([i1[:, None], i2[:, None]], axis=1)
        c_ref[...] = (jnp.where(cols == i1[:, None], p1n, 0.0)
                      + jnp.where(cols == i2[:, None], p2n, 0.0))

    def h_half(lo):
        hh = jax.lax.dot_general(
            x_ref[...], w1_ref[0],
            dimension_numbers=(((1,), (1,)), ((), ())),
            preferred_element_type=jnp.float32)
        hh = hh + b1_ref[0][:, lo:lo + FH]
        return 0.5 * hh * (1.0 + jax.lax.erf(hh * 0.7071067811865476))

    @pl.when(s == 0)
    def _a0():
        h_ref[0] = h_half(0)

    @pl.when(s == 1)
    def _a1():
        h_ref[1] = h_half(FH)

    def out_half(lo):
        cols = jax.lax.broadcasted_iota(jnp.int32, (T, E), 1)
        coef = jnp.sum(jnp.where(cols == e, c_ref[...], 0.0), axis=1,
                       keepdims=True)
        w2c = w2_ref[0]
        o = (jax.lax.dot_general(
                h_ref[0], w2c[:, 0:FH],
                dimension_numbers=(((1,), (1,)), ((), ())),
                preferred_element_type=jnp.float32)
             + jax.lax.dot_general(
                h_ref[1], w2c[:, FH:FFN],
                dimension_numbers=(((1,), (1,)), ((), ())),
                preferred_element_type=jnp.float32))
        contrib = coef * (o + b2_ref[0][:, lo:lo + HH])

        @pl.when(e == 0)
        def _first():
            out_ref[:, lo:lo + HH] = contrib

        @pl.when(e > 0)
        def _rest():
            out_ref[:, lo:lo + HH] += contrib

    @pl.when(s == 2)
    def _b0():
        out_half(0)

    @pl.when(s == 3)
    def _b1():
        out_half(HH)


@jax.jit
def kernel(x, Wr, W1, b1, W2, b2):
    B, S, _ = x.shape
    xf = x.reshape(T, H)

    out, probs, topk = pl.pallas_call(
        _moe_body,
        grid=(E, 4),
        in_specs=[
            pl.BlockSpec((T, H), lambda e, s: (0, 0)),
            pl.BlockSpec((E, H), lambda e, s: (0, 0)),
            pl.BlockSpec((1, FH, H), lambda e, s: (e, jnp.minimum(s, 1), 0)),
            pl.BlockSpec((1, 1, FFN), lambda e, s: (e, 0, 0)),
            pl.BlockSpec((1, HH, FFN), lambda e, s: (e, jnp.maximum(s - 2, 0), 0)),
            pl.BlockSpec((1, 1, H), lambda e, s: (e, 0, 0)),
        ],
        out_specs=[
            pl.BlockSpec((T, H), lambda e, s: (0, 0)),
            pl.BlockSpec((T, E), lambda e, s: (0, 0)),
            pl.BlockSpec((T, TOP_K), lambda e, s: (0, 0)),
        ],
        out_shape=[
            jax.ShapeDtypeStruct((T, H), jnp.float32),
            jax.ShapeDtypeStruct((T, E), jnp.float32),
            jax.ShapeDtypeStruct((T, TOP_K), jnp.int32),
        ],
        scratch_shapes=[
            pltpu.VMEM((T, E), jnp.float32),
            pltpu.VMEM((2, T, FH), jnp.float32),
        ],
        compiler_params=pltpu.CompilerParams(
            dimension_semantics=("arbitrary", "arbitrary"),
        ),
    )(xf, Wr, W1, b1.reshape(E, 1, FFN), W2, b2.reshape(E, 1, H))

    return (out.reshape(B, S, H), probs.reshape(B, S, E),
            topk.reshape(B, S, TOP_K))


# speedup vs baseline: 1.2616x; 1.2616x over previous
import jax
import jax.numpy as jnp
from jax.experimental import pallas as pl
from jax.experimental.pallas import tpu as pltpu

E = 16
H = 1024
FFN = 2048
TOP_K = 2
T = 128
FH = FFN // 2
HH = H // 2


def _moe_body(x_ref, wr_ref, w1a_ref, w1b_ref, b1_ref,
              w2a_ref, w2b_ref, b2_ref,
              out_ref, probs_ref, topk_ref, c_ref):
    e = pl.program_id(0)

    @pl.when(e == 0)
    def _router():
        xx = x_ref[...]
        g = jax.lax.dot_general(
            xx, wr_ref[...],
            dimension_numbers=(((1,), (1,)), ((), ())),
            preferred_element_type=jnp.float32)
        m = jnp.max(g, axis=-1, keepdims=True)
        ex = jnp.exp(g - m)
        p = ex / jnp.sum(ex, axis=-1, keepdims=True)
        probs_ref[...] = p
        cols = jax.lax.broadcasted_iota(jnp.int32, (T, E), 1)
        i1 = jnp.argmax(p, axis=-1)
        p1 = jnp.max(p, axis=-1)
        pm = jnp.where(cols == i1[:, None], -1.0, p)
        i2 = jnp.argmax(pm, axis=-1)
        p2 = jnp.max(pm, axis=-1)
        s = p1 + p2 + 1e-9
        p1n = (p1 / s)[:, None]
        p2n = (p2 / s)[:, None]
        topk_ref[...] = jnp.concatenate([i1[:, None], i2[:, None]], axis=1)
        c_ref[...] = (jnp.where(cols == i1[:, None], p1n, 0.0)
                      + jnp.where(cols == i2[:, None], p2n, 0.0))

    cols = jax.lax.broadcasted_iota(jnp.int32, (T, E), 1)
    coef = jnp.sum(jnp.where(cols == e, c_ref[...], 0.0), axis=1,
                   keepdims=True)

    def ffn_half(w1_half, b1_lo):
        hh = jax.lax.dot_general(
            x_ref[...], w1_half,
            dimension_numbers=(((1,), (1,)), ((), ())),
            preferred_element_type=jnp.float32)
        hh = hh + b1_ref[0][:, b1_lo:b1_lo + FH]
        return 0.5 * hh * (1.0 + jax.lax.erf(hh * 0.7071067811865476))

    h = jnp.concatenate(
        [ffn_half(w1a_ref[0], 0), ffn_half(w1b_ref[0], FH)], axis=1)

    o = jnp.concatenate(
        [jax.lax.dot_general(
            h, w2a_ref[0], dimension_numbers=(((1,), (1,)), ((), ())),
            preferred_element_type=jnp.float32),
         jax.lax.dot_general(
            h, w2b_ref[0], dimension_numbers=(((1,), (1,)), ((), ())),
            preferred_element_type=jnp.float32)], axis=1)
    contrib = coef * (o + b2_ref[0])

    @pl.when(e == 0)
    def _first():
        out_ref[...] = contrib

    @pl.when(e > 0)
    def _rest():
        out_ref[...] += contrib


@jax.jit
def kernel(x, Wr, W1, b1, W2, b2):
    B, S, _ = x.shape
    xf = x.reshape(T, H)

    out, probs, topk = pl.pallas_call(
        _moe_body,
        grid=(E,),
        in_specs=[
            pl.BlockSpec((T, H), lambda e: (0, 0)),
            pl.BlockSpec((E, H), lambda e: (0, 0)),
            pl.BlockSpec((1, FH, H), lambda e: (e, 0, 0)),
            pl.BlockSpec((1, FH, H), lambda e: (e, 1, 0)),
            pl.BlockSpec((1, 1, FFN), lambda e: (e, 0, 0)),
            pl.BlockSpec((1, HH, FFN), lambda e: (e, 0, 0)),
            pl.BlockSpec((1, HH, FFN), lambda e: (e, 1, 0)),
            pl.BlockSpec((1, 1, H), lambda e: (e, 0, 0)),
        ],
        out_specs=[
            pl.BlockSpec((T, H), lambda e: (0, 0)),
            pl.BlockSpec((T, E), lambda e: (0, 0)),
            pl.BlockSpec((T, TOP_K), lambda e: (0, 0)),
        ],
        out_shape=[
            jax.ShapeDtypeStruct((T, H), jnp.float32),
            jax.ShapeDtypeStruct((T, E), jnp.float32),
            jax.ShapeDtypeStruct((T, TOP_K), jnp.int32),
        ],
        scratch_shapes=[pltpu.VMEM((T, E), jnp.float32)],
        compiler_params=pltpu.CompilerParams(
            dimension_semantics=("arbitrary",),
        ),
    )(xf, Wr, W1, W1, b1.reshape(E, 1, FFN), W2, W2, b2.reshape(E, 1, H))

    return (out.reshape(B, S, H), probs.reshape(B, S, E),
            topk.reshape(B, S, TOP_K))


# speedup vs baseline: 1.2702x; 1.0068x over previous
import jax
import jax.numpy as jnp
from jax.experimental import pallas as pl
from jax.experimental.pallas import tpu as pltpu

E = 16
H = 1024
FFN = 2048
TOP_K = 2
T = 128
DEPTH = 3


def _moe_body(x_ref, wr_ref, w1_hbm, b1_ref, w2_hbm, b2_ref,
              out_ref, probs_ref, topk_ref,
              w1_buf, w2_buf, c_ref, sem1, sem2):
    def start_copy(e, slot):
        pltpu.make_async_copy(w1_hbm.at[e], w1_buf.at[slot],
                              sem1.at[slot]).start()
        pltpu.make_async_copy(w2_hbm.at[e], w2_buf.at[slot],
                              sem2.at[slot]).start()

    for k in range(DEPTH):
        start_copy(k, k)

    xx = x_ref[...]
    g = jax.lax.dot_general(
        xx, wr_ref[...],
        dimension_numbers=(((1,), (1,)), ((), ())),
        preferred_element_type=jnp.float32)
    m = jnp.max(g, axis=-1, keepdims=True)
    ex = jnp.exp(g - m)
    p = ex / jnp.sum(ex, axis=-1, keepdims=True)
    probs_ref[...] = p
    cols = jax.lax.broadcasted_iota(jnp.int32, (T, E), 1)
    i1 = jnp.argmax(p, axis=-1)
    p1 = jnp.max(p, axis=-1)
    pm = jnp.where(cols == i1[:, None], -1.0, p)
    i2 = jnp.argmax(pm, axis=-1)
    p2 = jnp.max(pm, axis=-1)
    s = p1 + p2 + 1e-9
    p1n = (p1 / s)[:, None]
    p2n = (p2 / s)[:, None]
    topk_ref[...] = jnp.concatenate([i1[:, None], i2[:, None]], axis=1)
    c_ref[...] = (jnp.where(cols == i1[:, None], p1n, 0.0)
                  + jnp.where(cols == i2[:, None], p2n, 0.0))
    out_ref[...] = jnp.zeros_like(out_ref)

    def step(e, carry):
        slot = jax.lax.rem(e, DEPTH)
        pltpu.make_async_copy(w1_hbm.at[e], w1_buf.at[slot],
                              sem1.at[slot]).wait()
        pltpu.make_async_copy(w2_hbm.at[e], w2_buf.at[slot],
                              sem2.at[slot]).wait()

        cols_ = jax.lax.broadcasted_iota(jnp.int32, (T, E), 1)
        coef = jnp.sum(jnp.where(cols_ == e, c_ref[...], 0.0), axis=1,
                       keepdims=True)
        h = jax.lax.dot_general(
            x_ref[...], w1_buf[slot],
            dimension_numbers=(((1,), (1,)), ((), ())),
            preferred_element_type=jnp.float32)
        h = h + b1_ref[e]
        h = 0.5 * h * (1.0 + jax.lax.erf(h * 0.7071067811865476))
        o = jax.lax.dot_general(
            h, w2_buf[slot],
            dimension_numbers=(((1,), (1,)), ((), ())),
            preferred_element_type=jnp.float32)
        out_ref[...] += coef * (o + b2_ref[e])

        @pl.when(e + DEPTH < E)
        def _next():
            start_copy(e + DEPTH, slot)

        return carry

    jax.lax.fori_loop(0, E, step, 0)


@jax.jit
def kernel(x, Wr, W1, b1, W2, b2):
    B, S, _ = x.shape
    xf = x.reshape(T, H)

    out, probs, topk = pl.pallas_call(
        _moe_body,
        in_specs=[
            pl.BlockSpec(memory_space=pltpu.VMEM),
            pl.BlockSpec(memory_space=pltpu.VMEM),
            pl.BlockSpec(memory_space=pltpu.HBM),
            pl.BlockSpec(memory_space=pltpu.VMEM),
            pl.BlockSpec(memory_space=pltpu.HBM),
            pl.BlockSpec(memory_space=pltpu.VMEM),
        ],
        out_specs=[
            pl.BlockSpec(memory_space=pltpu.VMEM),
            pl.BlockSpec(memory_space=pltpu.VMEM),
            pl.BlockSpec(memory_space=pltpu.VMEM),
        ],
        out_shape=[
            jax.ShapeDtypeStruct((T, H), jnp.float32),
            jax.ShapeDtypeStruct((T, E), jnp.float32),
            jax.ShapeDtypeStruct((T, TOP_K), jnp.int32),
        ],
        scratch_shapes=[
            pltpu.VMEM((DEPTH, FFN, H), jnp.float32),
            pltpu.VMEM((DEPTH, H, FFN), jnp.float32),
            pltpu.VMEM((T, E), jnp.float32),
            pltpu.SemaphoreType.DMA((DEPTH,)),
            pltpu.SemaphoreType.DMA((DEPTH,)),
        ],
        compiler_params=pltpu.CompilerParams(
            vmem_limit_bytes=100 * 1024 * 1024,
        ),
    )(xf, Wr, W1, b1.reshape(E, 1, FFN), W2, b2.reshape(E, 1, H))

    return (out.reshape(B, S, H), probs.reshape(B, S, E),
            topk.reshape(B, S, TOP_K))
